# Initial kernel scaffold; baseline (speedup 1.0000x reference)
import jax
import jax.numpy as jnp
from jax.experimental import pallas as pl


def kernel(x, pos, edge_index, W1m, b1m, g1m, be1m, W2m, b2m, Wa, ba, W1u, b1u, g1u, be1u, W2u, b2u, g_out, b_out):
    raise NotImplementedError("write your pallas kernel here")


# speedup vs baseline: 1.2433x; 1.2433x over previous
import math

import jax
import jax.numpy as jnp
from jax.experimental import pallas as pl
from jax.experimental.pallas import tpu as pltpu

B, V, E, D, PE, H = 1, 10000, 320000, 128, 32, 4
HD = D // H
MI = 2 * D + PE
EBLK = 512
NBLK = 1000


def _edge_kernel(xs_ref, xd_ref, ps_ref, pd_ref,
                 w1s_ref, w1d_ref, w1e_ref, b1_ref, g1_ref, be1_ref,
                 w2_ref, b2_ref, was_ref, wad_ref, wae_ref, ba_ref,
                 wm_ref, ex_ref):
    xs = xs_ref[...]
    xd = xd_ref[...]
    rel = ps_ref[...] - pd_ref[...]
    half = PE // 4
    scale = math.log(10.0) / (half - 1)
    ks = jax.lax.broadcasted_iota(jnp.int32, (1, half), 1).astype(jnp.float32)
    freqs = jnp.exp(ks * -scale)
    xe = rel[:, 0:1] * freqs
    ye = rel[:, 1:2] * freqs
    enc = jnp.concatenate(
        [jnp.sin(xe), jnp.cos(xe), jnp.sin(ye), jnp.cos(ye)], axis=-1)
    h1 = (jnp.dot(xs, w1s_ref[...], preferred_element_type=jnp.float32)
          + jnp.dot(xd, w1d_ref[...], preferred_element_type=jnp.float32)
          + jnp.dot(enc, w1e_ref[...], preferred_element_type=jnp.float32)
          + b1_ref[...])
    m = jnp.mean(h1, axis=-1, keepdims=True)
    v = jnp.mean((h1 - m) * (h1 - m), axis=-1, keepdims=True)
    hn = (h1 - m) * jax.lax.rsqrt(v + 1e-5) * g1_ref[...] + be1_ref[...]
    h = hn * jax.nn.sigmoid(hn)
    msg = jnp.dot(h, w2_ref[...], preferred_element_type=jnp.float32) + b2_ref[...]
    logits = (jnp.dot(xs, was_ref[...], preferred_element_type=jnp.float32)
              + jnp.dot(xd, wad_ref[...], preferred_element_type=jnp.float32)
              + jnp.dot(enc, wae_ref[...], preferred_element_type=jnp.float32)
              + ba_ref[...])
    ex = jnp.exp(logits)
    exb = jnp.repeat(ex, HD, axis=1)
    wm_ref[...] = exb * msg
    ex_ref[...] = ex


def _node_kernel(x_ref, num_ref, den_ref,
                 w1x_ref, w1a_ref, b1_ref, g1_ref, be1_ref,
                 w2_ref, b2_ref, go_ref, bo_ref, out_ref):
    x = x_ref[...]
    den = den_ref[...]
    agg = num_ref[...] / (jnp.repeat(den, HD, axis=1) + 1e-8)
    h1 = (jnp.dot(x, w1x_ref[...], preferred_element_type=jnp.float32)
          + jnp.dot(agg, w1a_ref[...], preferred_element_type=jnp.float32)
          + b1_ref[...])
    m = jnp.mean(h1, axis=-1, keepdims=True)
    v = jnp.mean((h1 - m) * (h1 - m), axis=-1, keepdims=True)
    hn = (h1 - m) * jax.lax.rsqrt(v + 1e-5) * g1_ref[...] + be1_ref[...]
    h2 = hn * jax.nn.sigmoid(hn)
    upd = jnp.dot(h2, w2_ref[...], preferred_element_type=jnp.float32) + b2_ref[...]
    y = x + upd
    m2 = jnp.mean(y, axis=-1, keepdims=True)
    v2 = jnp.mean((y - m2) * (y - m2), axis=-1, keepdims=True)
    out_ref[...] = (y - m2) * jax.lax.rsqrt(v2 + 1e-5) * go_ref[...] + bo_ref[...]


def _rep(shape):
    nd = len(shape)
    return pl.BlockSpec(shape, lambda i: (0,) * nd)


def kernel(x, pos, edge_index, W1m, b1m, g1m, be1m, W2m, b2m, Wa, ba,
           W1u, b1u, g1u, be1u, W2u, b2u, g_out, b_out):
    src = edge_index[0]
    dst = edge_index[1]
    x2 = x[0]
    pos2 = pos[0]
    xs = x2[src]
    xd = x2[dst]
    ps = pos2[src]
    pd = pos2[dst]

    grid_e = E // EBLK
    eb = lambda w: pl.BlockSpec((EBLK, w), lambda i: (i, 0))
    wm, ex = pl.pallas_call(
        _edge_kernel,
        grid=(grid_e,),
        in_specs=[eb(D), eb(D), eb(2), eb(2),
                  _rep((D, 2 * D)), _rep((D, 2 * D)), _rep((PE, 2 * D)),
                  _rep((2 * D,)), _rep((2 * D,)), _rep((2 * D,)),
                  _rep((2 * D, D)), _rep((D,)),
                  _rep((D, H)), _rep((D, H)), _rep((PE, H)), _rep((H,))],
        out_specs=[eb(D), eb(H)],
        out_shape=[jax.ShapeDtypeStruct((E, D), jnp.float32),
                   jax.ShapeDtypeStruct((E, H), jnp.float32)],
    )(xs, xd, ps, pd,
      W1m[:D], W1m[D:2 * D], W1m[2 * D:], b1m, g1m, be1m,
      W2m, b2m, Wa[:D], Wa[D:2 * D], Wa[2 * D:], ba)

    packed = jnp.concatenate([wm, ex], axis=-1)
    tab = jax.ops.segment_sum(packed, dst, num_segments=V)
    num = tab[:, :D]
    den = tab[:, D:]

    grid_n = V // NBLK
    nb = lambda w: pl.BlockSpec((NBLK, w), lambda i: (i, 0))
    out = pl.pallas_call(
        _node_kernel,
        grid=(grid_n,),
        in_specs=[nb(D), nb(D), nb(H),
                  _rep((D, 2 * D)), _rep((D, 2 * D)),
                  _rep((2 * D,)), _rep((2 * D,)), _rep((2 * D,)),
                  _rep((2 * D, D)), _rep((D,)), _rep((D,)), _rep((D,))],
        out_specs=nb(D),
        out_shape=jax.ShapeDtypeStruct((V, D), jnp.float32),
    )(x2, num, den,
      W1u[:D], W1u[D:], b1u, g1u, be1u, W2u, b2u, g_out, b_out)
    return out[None]


# speedup vs baseline: 2.1858x; 1.7581x over previous
import math

import jax
import jax.numpy as jnp
from jax import lax
from jax.experimental import pallas as pl
from jax.experimental.pallas import tpu as pltpu
from jax.experimental.pallas import tpu_sc as plsc

B, V, E, D, PE, H = 1, 10000, 320000, 128, 32, 4
HD = D // H
MI = 2 * D + PE
EBLK = 512
NBLK = 1000

_NC, _NS = 2, 16
_NW = _NC * _NS
_EPW = E // _NW
_C = 400
_NCH = _EPW // _C
_NG = _C // 16


def _gather_kernel(x_hbm, p_hbm, src_hbm, dst_hbm,
                   xs_out, xd_out, relx_out, rely_out,
                   idx_s, idx_d, i2xs, i2ys, i2xd, i2yd,
                   pxs_v, pys_v, pxd_v, pyd_v, relx_v, rely_v,
                   xs_v, xd_v, sem0, sem1, sem2):
    wid = lax.axis_index("s") * _NC + lax.axis_index("c")

    def body(i, carry):
        base = wid * _EPW + i * _C
        i0 = pltpu.async_copy(src_hbm.at[pl.ds(base, _C)], idx_s, sem0)
        i1 = pltpu.async_copy(dst_hbm.at[pl.ds(base, _C)], idx_d, sem1)
        i0.wait()
        i1.wait()
        g0 = pltpu.async_copy(x_hbm.at[idx_s], xs_v, sem0)
        g1 = pltpu.async_copy(x_hbm.at[idx_d], xd_v, sem1)
        for g in range(_NG):
            sl = pl.ds(g * 16, 16)
            iv_s = idx_s[sl] * 2
            iv_d = idx_d[sl] * 2
            i2xs[sl] = iv_s
            i2ys[sl] = iv_s + 1
            i2xd[sl] = iv_d
            i2yd[sl] = iv_d + 1
        p0 = pltpu.async_copy(p_hbm.at[i2xs], pxs_v, sem2)
        p1 = pltpu.async_copy(p_hbm.at[i2ys], pys_v, sem2)
        p2 = pltpu.async_copy(p_hbm.at[i2xd], pxd_v, sem2)
        p3 = pltpu.async_copy(p_hbm.at[i2yd], pyd_v, sem2)
        p0.wait()
        p1.wait()
        p2.wait()
        p3.wait()
        for g in range(_NG):
            sl = pl.ds(g * 16, 16)
            relx_v[sl] = pxs_v[sl] - pxd_v[sl]
            rely_v[sl] = pys_v[sl] - pyd_v[sl]
        r0 = pltpu.async_copy(relx_v, relx_out.at[pl.ds(base, _C)], sem2)
        r1 = pltpu.async_copy(rely_v, rely_out.at[pl.ds(base, _C)], sem2)
        g0.wait()
        g1.wait()
        o0 = pltpu.async_copy(xs_v, xs_out.at[pl.ds(base, _C)], sem0)
        o1 = pltpu.async_copy(xd_v, xd_out.at[pl.ds(base, _C)], sem1)
        r0.wait()
        r1.wait()
        o0.wait()
        o1.wait()
        return carry

    lax.fori_loop(0, _NCH, body, 0)


def _edge_kernel(xs_ref, xd_ref, rel_ref,
                 w1s_ref, w1d_ref, w1e_ref, b1_ref, g1_ref, be1_ref,
                 w2_ref, b2_ref, was_ref, wad_ref, wae_ref, ba_ref,
                 out_ref):
    xs = xs_ref[...]
    xd = xd_ref[...]
    rel = rel_ref[...]
    half = PE // 4
    scale = math.log(10.0) / (half - 1)
    ks = lax.broadcasted_iota(jnp.int32, (1, half), 1).astype(jnp.float32)
    freqs = jnp.exp(ks * -scale)
    xe = rel[:, 0:1] * freqs
    ye = rel[:, 1:2] * freqs
    enc = jnp.concatenate(
        [jnp.sin(xe), jnp.cos(xe), jnp.sin(ye), jnp.cos(ye)], axis=-1)
    h1 = (jnp.dot(xs, w1s_ref[...], preferred_element_type=jnp.float32)
          + jnp.dot(xd, w1d_ref[...], preferred_element_type=jnp.float32)
          + jnp.dot(enc, w1e_ref[...], preferred_element_type=jnp.float32)
          + b1_ref[...])
    m = jnp.mean(h1, axis=-1, keepdims=True)
    v = jnp.mean((h1 - m) * (h1 - m), axis=-1, keepdims=True)
    hn = (h1 - m) * lax.rsqrt(v + 1e-5) * g1_ref[...] + be1_ref[...]
    h = hn * jax.nn.sigmoid(hn)
    msg = jnp.dot(h, w2_ref[...], preferred_element_type=jnp.float32) + b2_ref[...]
    logits = (jnp.dot(xs, was_ref[...], preferred_element_type=jnp.float32)
              + jnp.dot(xd, wad_ref[...], preferred_element_type=jnp.float32)
              + jnp.dot(enc, wae_ref[...], preferred_element_type=jnp.float32)
              + ba_ref[...])
    ex = jnp.exp(logits)
    exb = jnp.repeat(ex, HD, axis=1)
    out_ref[:, :D] = exb * msg
    out_ref[:, D:] = jnp.concatenate(
        [ex, jnp.zeros((EBLK, 12), jnp.float32)], axis=-1)


def _node_kernel(x_ref, num_ref, den_ref,
                 w1x_ref, w1a_ref, b1_ref, g1_ref, be1_ref,
                 w2_ref, b2_ref, go_ref, bo_ref, out_ref):
    x = x_ref[...]
    den = den_ref[:, :H]
    agg = num_ref[...] / (jnp.repeat(den, HD, axis=1) + 1e-8)
    h1 = (jnp.dot(x, w1x_ref[...], preferred_element_type=jnp.float32)
          + jnp.dot(agg, w1a_ref[...], preferred_element_type=jnp.float32)
          + b1_ref[...])
    m = jnp.mean(h1, axis=-1, keepdims=True)
    v = jnp.mean((h1 - m) * (h1 - m), axis=-1, keepdims=True)
    hn = (h1 - m) * lax.rsqrt(v + 1e-5) * g1_ref[...] + be1_ref[...]
    h2 = hn * jax.nn.sigmoid(hn)
    upd = jnp.dot(h2, w2_ref[...], preferred_element_type=jnp.float32) + b2_ref[...]
    y = x + upd
    m2 = jnp.mean(y, axis=-1, keepdims=True)
    v2 = jnp.mean((y - m2) * (y - m2), axis=-1, keepdims=True)
    out_ref[...] = (y - m2) * lax.rsqrt(v2 + 1e-5) * go_ref[...] + bo_ref[...]


def _rep(shape):
    nd = len(shape)
    return pl.BlockSpec(shape, lambda i: (0,) * nd)


def kernel(x, pos, edge_index, W1m, b1m, g1m, be1m, W2m, b2m, Wa, ba,
           W1u, b1u, g1u, be1u, W2u, b2u, g_out, b_out):
    src = edge_index[0]
    dst = edge_index[1]
    x2 = x[0]
    pflat = pos[0].reshape(2 * V)

    mesh = plsc.VectorSubcoreMesh(core_axis_name="c", subcore_axis_name="s")
    gather = pl.kernel(
        _gather_kernel,
        mesh=mesh,
        out_type=[jax.ShapeDtypeStruct((E, D), jnp.float32),
                  jax.ShapeDtypeStruct((E, D), jnp.float32),
                  jax.ShapeDtypeStruct((E,), jnp.float32),
                  jax.ShapeDtypeStruct((E,), jnp.float32)],
        scratch_types=([pltpu.VMEM((_C,), jnp.int32)] * 6
                       + [pltpu.VMEM((_C,), jnp.float32)] * 6
                       + [pltpu.VMEM((_C, D), jnp.float32)] * 2
                       + [pltpu.SemaphoreType.DMA] * 3),
    )
    xs_g, xd_g, relx_g, rely_g = gather(x2, pflat, src, dst)
    rel_g = jnp.stack([relx_g, rely_g], axis=-1)

    grid_e = E // EBLK
    eb = lambda w: pl.BlockSpec((EBLK, w), lambda i: (i, 0))
    packed = pl.pallas_call(
        _edge_kernel,
        grid=(grid_e,),
        in_specs=[eb(D), eb(D), eb(2),
                  _rep((D, 2 * D)), _rep((D, 2 * D)), _rep((PE, 2 * D)),
                  _rep((2 * D,)), _rep((2 * D,)), _rep((2 * D,)),
                  _rep((2 * D, D)), _rep((D,)),
                  _rep((D, H)), _rep((D, H)), _rep((PE, H)), _rep((H,))],
        out_specs=eb(D + 16),
        out_shape=jax.ShapeDtypeStruct((E, D + 16), jnp.float32),
    )(xs_g, xd_g, rel_g,
      W1m[:D], W1m[D:2 * D], W1m[2 * D:], b1m, g1m, be1m,
      W2m, b2m, Wa[:D], Wa[D:2 * D], Wa[2 * D:], ba)

    tab = jax.ops.segment_sum(packed, dst, num_segments=V)

    grid_n = V // NBLK
    nb = lambda w: pl.BlockSpec((NBLK, w), lambda i: (i, 0))
    out = pl.pallas_call(
        _node_kernel,
        grid=(grid_n,),
        in_specs=[nb(D), nb(D), nb(16),
                  _rep((D, 2 * D)), _rep((D, 2 * D)),
                  _rep((2 * D,)), _rep((2 * D,)), _rep((2 * D,)),
                  _rep((2 * D, D)), _rep((D,)), _rep((D,)), _rep((D,))],
        out_specs=nb(D),
        out_shape=jax.ShapeDtypeStruct((V, D), jnp.float32),
    )(x2, tab[:, :D], tab[:, D:],
      W1u[:D], W1u[D:], b1u, g1u, be1u, W2u, b2u, g_out, b_out)
    return out[None]


# speedup vs baseline: 2.6198x; 1.1986x over previous
import math

import jax
import jax.numpy as jnp
from jax import lax
from jax.experimental import pallas as pl
from jax.experimental.pallas import tpu as pltpu
from jax.experimental.pallas import tpu_sc as plsc

B, V, E, D, PE, H = 1, 10000, 320000, 128, 32, 4
HD = D // H
MI = 2 * D + PE
EBLK = 512
NBLK = 1000

_NC, _NS = 2, 16
_NW = _NC * _NS
_EPW = E // _NW
_C = 400
_NCH = _EPW // _C
_NG = _C // 16


def _gather_kernel(x_hbm, p_hbm, src_hbm, dst_hbm,
                   xs_out, xd_out, relx_out, rely_out,
                   idx_s, idx_d, i2xs, i2ys, i2xd, i2yd,
                   pxs_v, pys_v, pxd_v, pyd_v, relx_v, rely_v,
                   xs_v, xd_v, sem0, sem1, sem2):
    wid = lax.axis_index("s") * _NC + lax.axis_index("c")

    def body(i, carry):
        base = wid * _EPW + i * _C
        i0 = pltpu.async_copy(src_hbm.at[pl.ds(base, _C)], idx_s, sem0)
        i1 = pltpu.async_copy(dst_hbm.at[pl.ds(base, _C)], idx_d, sem1)
        i0.wait()
        i1.wait()
        g0 = pltpu.async_copy(x_hbm.at[idx_s], xs_v, sem0)
        g1 = pltpu.async_copy(x_hbm.at[idx_d], xd_v, sem1)
        for g in range(_NG):
            sl = pl.ds(g * 16, 16)
            iv_s = idx_s[sl] * 2
            iv_d = idx_d[sl] * 2
            i2xs[sl] = iv_s
            i2ys[sl] = iv_s + 1
            i2xd[sl] = iv_d
            i2yd[sl] = iv_d + 1
        p0 = pltpu.async_copy(p_hbm.at[i2xs], pxs_v, sem2)
        p1 = pltpu.async_copy(p_hbm.at[i2ys], pys_v, sem2)
        p2 = pltpu.async_copy(p_hbm.at[i2xd], pxd_v, sem2)
        p3 = pltpu.async_copy(p_hbm.at[i2yd], pyd_v, sem2)
        p0.wait()
        p1.wait()
        p2.wait()
        p3.wait()
        for g in range(_NG):
            sl = pl.ds(g * 16, 16)
            relx_v[sl] = pxs_v[sl] - pxd_v[sl]
            rely_v[sl] = pys_v[sl] - pyd_v[sl]
        r0 = pltpu.async_copy(relx_v, relx_out.at[pl.ds(base, _C)], sem2)
        r1 = pltpu.async_copy(rely_v, rely_out.at[pl.ds(base, _C)], sem2)
        g0.wait()
        g1.wait()
        o0 = pltpu.async_copy(xs_v, xs_out.at[pl.ds(base, _C)], sem0)
        o1 = pltpu.async_copy(xd_v, xd_out.at[pl.ds(base, _C)], sem1)
        r0.wait()
        r1.wait()
        o0.wait()
        o1.wait()
        return carry

    lax.fori_loop(0, _NCH, body, 0)


def _bdot(a, b):
    return lax.dot_general(a, b, (((1,), (0,)), ((), ())),
                           preferred_element_type=jnp.float32)


def _split(a):
    ah = a.astype(jnp.bfloat16)
    al = (a - ah.astype(jnp.float32)).astype(jnp.bfloat16)
    return ah, al


def _dot3(a, bh, bl):
    ah, al = _split(a)
    return _bdot(ah, bh) + _bdot(ah, bl) + _bdot(al, bh)

def _edge_kernel(xs_ref, xd_ref, rel_ref, fx_ref, fy_ref, off_ref,
                 w1sh_ref, w1sl_ref, w1dh_ref, w1dl_ref, w1eh_ref, w1el_ref,
                 b1_ref, g1_ref, be1_ref,
                 w2h_ref, w2l_ref, b2_ref, was_ref, wad_ref, wae_ref, ba_ref,
                 out_ref):
    xs = xs_ref[...]
    xd = xd_ref[...]
    rel = rel_ref[...]
    phase = (rel[:, 0:1] * fx_ref[...] + rel[:, 1:2] * fy_ref[...]
             + off_ref[...])
    enc = jnp.sin(phase)
    h1 = (_dot3(xs, w1sh_ref[...], w1sl_ref[...])
          + _dot3(xd, w1dh_ref[...], w1dl_ref[...])
          + _dot3(enc, w1eh_ref[...], w1el_ref[...])
          + b1_ref[...])
    m = jnp.mean(h1, axis=-1, keepdims=True)
    v = jnp.mean((h1 - m) * (h1 - m), axis=-1, keepdims=True)
    hn = (h1 - m) * lax.rsqrt(v + 1e-5) * g1_ref[...] + be1_ref[...]
    h = hn * jax.nn.sigmoid(hn)
    msg = _dot3(h, w2h_ref[...], w2l_ref[...]) + b2_ref[...]
    logits_rep = (_bdot(xs.astype(jnp.bfloat16), was_ref[...])
                  + _bdot(xd.astype(jnp.bfloat16), wad_ref[...])
                  + _bdot(enc.astype(jnp.bfloat16), wae_ref[...])
                  + ba_ref[...])
    exb = jnp.exp(logits_rep)
    out_ref[:, :D] = exb * msg
    out_ref[:, D:] = jnp.concatenate(
        [exb[:, 0:4], jnp.zeros((EBLK, 12), jnp.float32)], axis=-1)


def _node_kernel(x_ref, num_ref, den_ref,
                 w1x_ref, w1a_ref, b1_ref, g1_ref, be1_ref,
                 w2_ref, b2_ref, go_ref, bo_ref, out_ref):
    x = x_ref[...]
    agg = num_ref[...] / (den_ref[...] + 1e-8)
    h1 = (jnp.dot(x, w1x_ref[...], preferred_element_type=jnp.float32)
          + jnp.dot(agg, w1a_ref[...], preferred_element_type=jnp.float32)
          + b1_ref[...])
    m = jnp.mean(h1, axis=-1, keepdims=True)
    v = jnp.mean((h1 - m) * (h1 - m), axis=-1, keepdims=True)
    hn = (h1 - m) * lax.rsqrt(v + 1e-5) * g1_ref[...] + be1_ref[...]
    h2 = hn * jax.nn.sigmoid(hn)
    upd = jnp.dot(h2, w2_ref[...], preferred_element_type=jnp.float32) + b2_ref[...]
    y = x + upd
    m2 = jnp.mean(y, axis=-1, keepdims=True)
    v2 = jnp.mean((y - m2) * (y - m2), axis=-1, keepdims=True)
    out_ref[...] = (y - m2) * lax.rsqrt(v2 + 1e-5) * go_ref[...] + bo_ref[...]


def _rep(shape):
    nd = len(shape)
    return pl.BlockSpec(shape, lambda i: (0,) * nd)


def _wsplit(w):
    wh = w.astype(jnp.bfloat16)
    wl = (w - wh.astype(jnp.float32)).astype(jnp.bfloat16)
    return wh, wl

def kernel(x, pos, edge_index, W1m, b1m, g1m, be1m, W2m, b2m, Wa, ba,
           W1u, b1u, g1u, be1u, W2u, b2u, g_out, b_out):
    src = edge_index[0]
    dst = edge_index[1]
    x2 = x[0]
    pflat = pos[0].reshape(2 * V)

    mesh = plsc.VectorSubcoreMesh(core_axis_name="c", subcore_axis_name="s")
    gather = pl.kernel(
        _gather_kernel,
        mesh=mesh,
        out_type=[jax.ShapeDtypeStruct((E, D), jnp.float32),
                  jax.ShapeDtypeStruct((E, D), jnp.float32),
                  jax.ShapeDtypeStruct((E,), jnp.float32),
                  jax.ShapeDtypeStruct((E,), jnp.float32)],
        scratch_types=([pltpu.VMEM((_C,), jnp.int32)] * 6
                       + [pltpu.VMEM((_C,), jnp.float32)] * 6
                       + [pltpu.VMEM((_C, D), jnp.float32)] * 2
                       + [pltpu.SemaphoreType.DMA] * 3),
    )
    xs_g, xd_g, relx_g, rely_g = gather(x2, pflat, src, dst)
    rel_g = jnp.stack([relx_g, rely_g], axis=-1)

    grid_e = E // EBLK
    eb = lambda w: pl.BlockSpec((EBLK, w), lambda i: (i, 0))
    packed_call = pl.pallas_call(
        _edge_kernel,
        grid=(grid_e,),
        in_specs=[eb(D), eb(D), eb(2),
                  _rep((D,)), _rep((D,)), _rep((D,)),
                  _rep((D, 2 * D)), _rep((D, 2 * D)),
                  _rep((D, 2 * D)), _rep((D, 2 * D)),
                  _rep((D, 2 * D)), _rep((D, 2 * D)),
                  _rep((2 * D,)), _rep((2 * D,)), _rep((2 * D,)),
                  _rep((2 * D, D)), _rep((2 * D, D)), _rep((D,)),
                  _rep((D, D)), _rep((D, D)), _rep((D, D)), _rep((D,))],
        out_specs=eb(D + 16),
        out_shape=jax.ShapeDtypeStruct((E, D + 16), jnp.float32),
    )
    half = PE // 4
    fscale = math.log(10.0) / (half - 1)
    freqs = jnp.exp(jnp.arange(half, dtype=jnp.float32) * -fscale)
    f32v = jnp.tile(freqs, 4)
    grp = jnp.arange(PE) // half
    fx = jnp.pad(jnp.where(grp < 2, f32v, 0.0), (0, D - PE))
    fy = jnp.pad(jnp.where(grp >= 2, f32v, 0.0), (0, D - PE))
    off = jnp.pad(jnp.where(grp % 2 == 1, 0.5 * jnp.pi, 0.0), (0, D - PE))
    wpad = lambda w: jnp.pad(w, ((0, D - PE), (0, 0)))
    w1sh, w1sl = _wsplit(W1m[:D])
    w1dh, w1dl = _wsplit(W1m[D:2 * D])
    w1eh, w1el = _wsplit(wpad(W1m[2 * D:]))
    perm = (jnp.arange(D) % H) * HD + jnp.arange(D) // H
    w2h, w2l = _wsplit(W2m[:, perm])
    packed = packed_call(
        xs_g, xd_g, rel_g, fx, fy, off,
        w1sh, w1sl, w1dh, w1dl, w1eh, w1el, b1m, g1m, be1m,
        w2h, w2l, b2m,
        jnp.tile(Wa[:D], (1, HD)).astype(jnp.bfloat16),
        jnp.tile(Wa[D:2 * D], (1, HD)).astype(jnp.bfloat16),
        wpad(jnp.tile(Wa[2 * D:], (1, HD))).astype(jnp.bfloat16),
        jnp.tile(ba, HD))

    tab = jax.ops.segment_sum(packed, dst, num_segments=V)

    grid_n = V // NBLK
    nb = lambda w: pl.BlockSpec((NBLK, w), lambda i: (i, 0))
    out_call = pl.pallas_call(
        _node_kernel,
        grid=(grid_n,),
        in_specs=[nb(D), nb(D), nb(D),
                  _rep((D, 2 * D)), _rep((D, 2 * D)),
                  _rep((2 * D,)), _rep((2 * D,)), _rep((2 * D,)),
                  _rep((2 * D, D)), _rep((D,)), _rep((D,)), _rep((D,))],
        out_specs=nb(D),
        out_shape=jax.ShapeDtypeStruct((V, D), jnp.float32),
    )
    den128 = jnp.tile(tab[:, D:D + H], (1, HD))
    out = out_call(x2, tab[:, :D], den128,
      W1u[:D], W1u[D:][perm], b1u, g1u, be1u, W2u, b2u, g_out, b_out)
    return out[None]
